# baseline (speedup 1.0000x reference)
import functools

import jax
import jax.numpy as jnp
from jax import lax
from jax.experimental import pallas as pl
from jax.experimental.pallas import tpu as pltpu
from jax.experimental.pallas import tpu_sc as plsc

N_FIELDS = 26
VOCAB = 100000
EMB = 32
BATCH = 4096
DENSE = 256
EMB_FLAT = N_FIELDS * EMB

NF_H = 13
NF_HP = 16
GROUPS_H = 4
HALF_COLS = NF_H * EMB

ROWS_H = NF_HP * BATCH
NUM_WORKERS = 32
ROWS_PER_WORKER = ROWS_H // NUM_WORKERS
CHUNK = 128
NCHUNKS = ROWS_PER_WORKER // CHUNK
OUT_ROWS = ROWS_H // 4
ORPC = CHUNK // 4


@functools.cache
def _get_sc_gather():
    mesh = plsc.VectorSubcoreMesh(core_axis_name="c", subcore_axis_name="s")

    @functools.partial(
        pl.kernel,
        out_type=jax.ShapeDtypeStruct((OUT_ROWS, 128), jnp.float32),
        mesh=mesh,
        scratch_types=[
            pltpu.VMEM((NCHUNKS, CHUNK), jnp.int32),
            pltpu.VMEM((CHUNK, EMB), jnp.float32),
            pltpu.VMEM((CHUNK, EMB), jnp.float32),
            pltpu.VMEM((ORPC, 128), jnp.float32),
            pltpu.SemaphoreType.DMA,
            pltpu.SemaphoreType.DMA,
        ],
        compiler_params=pltpu.CompilerParams(use_tc_tiling_on_sc=False),
    )
    def _sc_gather(table_hbm, idx_hbm, out_hbm, idx_v, buf0, buf1, cbuf,
                   sem0, sem1):
        wid = lax.axis_index("s") * 2 + lax.axis_index("c")
        base = wid * (ROWS_PER_WORKER // 4)
        pltpu.sync_copy(idx_hbm.at[wid], idx_v)

        bufs = (buf0, buf1)
        sems = (sem0, sem1)

        def start(j, b):
            return pltpu.async_copy(
                table_hbm.at[idx_v.at[j]], bufs[b], sems[b]
            )

        start(0, 0)

        def body(t, carry):
            for b in range(2):
                j = 2 * t + b

                @pl.when(j + 1 < NCHUNKS)
                def _():
                    start(j + 1, 1 - b)

                pltpu.make_async_copy(
                    table_hbm.at[idx_v.at[j]], bufs[b], sems[b]
                ).wait()
                buf = bufs[b]
                for k in range(ORPC):
                    for jj in range(4):
                        for m in range(2):
                            cbuf[k, pl.ds(32 * jj + 16 * m, 16)] = (
                                buf[4 * k + jj, pl.ds(16 * m, 16)]
                            )
                pltpu.sync_copy(
                    cbuf, out_hbm.at[pl.ds(base + j * ORPC, ORPC)]
                )
            return carry

        lax.fori_loop(0, NCHUNKS // 2, body, 0)

    return _sc_gather


def _mlp_body(elo_ref, ehi_ref, xd_ref, w1lo_ref, w1hi_ref, w1b_ref,
              b1_ref, w2_ref, b2_ref, w3_ref, b3_ref, out_ref):
    h = jnp.dot(elo_ref[0], w1lo_ref[0], preferred_element_type=jnp.float32)
    for g in range(1, GROUPS_H):
        h = h + jnp.dot(elo_ref[g], w1lo_ref[g],
                        preferred_element_type=jnp.float32)
    for g in range(GROUPS_H):
        h = h + jnp.dot(ehi_ref[g], w1hi_ref[g],
                        preferred_element_type=jnp.float32)
    h = h + jnp.dot(xd_ref[...], w1b_ref[...],
                    preferred_element_type=jnp.float32)
    h = jnp.maximum(h + b1_ref[...], 0.0)
    h = jnp.dot(h, w2_ref[...], preferred_element_type=jnp.float32)
    h = jnp.maximum(h + b2_ref[...], 0.0)
    out_ref[...] = (
        jnp.dot(h, w3_ref[...], preferred_element_type=jnp.float32)
        + b3_ref[...]
    )


_BM = 1024


def _mlp(elo, ehi, xd, w1lo, w1hi, w1b, b1, w2, b2, w3, b3):
    grid = (BATCH // _BM,)
    return pl.pallas_call(
        _mlp_body,
        grid=grid,
        in_specs=[
            pl.BlockSpec((GROUPS_H, _BM, 128), lambda i: (0, i, 0)),
            pl.BlockSpec((GROUPS_H, _BM, 128), lambda i: (0, i, 0)),
            pl.BlockSpec((_BM, DENSE), lambda i: (i, 0)),
            pl.BlockSpec((GROUPS_H, 128, 128), lambda i: (0, 0, 0)),
            pl.BlockSpec((GROUPS_H, 128, 128), lambda i: (0, 0, 0)),
            pl.BlockSpec((DENSE, 128), lambda i: (0, 0)),
            pl.BlockSpec((128,), lambda i: (0,)),
            pl.BlockSpec((128, 64), lambda i: (0, 0)),
            pl.BlockSpec((64,), lambda i: (0,)),
            pl.BlockSpec((64, 1), lambda i: (0, 0)),
            pl.BlockSpec((1,), lambda i: (0,)),
        ],
        out_specs=pl.BlockSpec((_BM, 1), lambda i: (i, 0)),
        out_shape=jax.ShapeDtypeStruct((BATCH, 1), jnp.float32),
    )(elo, ehi, xd, w1lo, w1hi, w1b, b1, w2, b2, w3, b3)


def _half_idx(xc_half):
    flat = jnp.concatenate(
        [xc_half, jnp.zeros((NF_HP - NF_H, BATCH), jnp.int32)], axis=0
    )
    idx = jnp.transpose(flat.reshape(GROUPS_H, 4, BATCH), (0, 2, 1))
    return idx.reshape(NUM_WORKERS, NCHUNKS, CHUNK)


def _w1_half(w1_half_cols):
    return jnp.concatenate(
        [w1_half_cols,
         jnp.zeros((NF_HP * EMB - HALF_COLS, 128), w1_half_cols.dtype)],
        axis=0,
    ).reshape(GROUPS_H, 128, 128)


def kernel(x_cat, x_dense, emb_table, W1, b1, W2, b2, W3, b3):
    xc = x_cat.astype(jnp.int32)
    offs = (jnp.arange(NF_H, dtype=jnp.int32) * VOCAB)[:, None]
    gather = _get_sc_gather()

    tlo = emb_table[:NF_H].reshape(NF_H * VOCAB, EMB)
    thi = emb_table[NF_H:].reshape(NF_H * VOCAB, EMB)
    rlo = gather(tlo, _half_idx(xc[:NF_H] + offs))
    rhi = gather(thi, _half_idx(xc[NF_H:] + offs))
    elo = rlo.reshape(GROUPS_H, BATCH, 128)
    ehi = rhi.reshape(GROUPS_H, BATCH, 128)

    out = _mlp(elo, ehi, x_dense,
               _w1_half(W1[:HALF_COLS]), _w1_half(W1[HALF_COLS:EMB_FLAT]),
               W1[EMB_FLAT:], b1, W2, b2, W3, b3)
    return out.reshape(BATCH)
